# Initial kernel scaffold; baseline (speedup 1.0000x reference)
import jax
import jax.numpy as jnp
from jax.experimental import pallas as pl


def kernel(x, edge_index, w1_0, b1_0, w1_1, b1_1, w2_0, b2_0, w2_1, b2_1, wfc, bfc):
    raise NotImplementedError("write your pallas kernel here")


# speedup vs baseline: 6.3117x; 6.3117x over previous
import functools

import jax
import jax.numpy as jnp
from jax import lax
from jax.experimental import pallas as pl
from jax.experimental.pallas import tpu as pltpu
from jax.experimental.pallas import tpu_sc as plsc

_N = 10000
_E = 320000
_D = 128
_H = 64
_C = 64

_NCORES = 2
_NSUB = 16
_NTILES = _NCORES * _NSUB
_CHUNK = 128
_CPT = 79
_EPAD = _NTILES * _CPT * _CHUNK
_ACC_ROWS = 10112
_ZROWS = _ACC_ROWS // _NSUB

_sc_mesh = plsc.VectorSubcoreMesh(core_axis_name="c", subcore_axis_name="s")


@functools.partial(
    pl.kernel,
    out_type=jax.ShapeDtypeStruct((_NCORES, _ACC_ROWS, _H), jnp.float32),
    mesh=_sc_mesh,
    scratch_types=[
        pltpu.VMEM((_CPT, _CHUNK), jnp.int32),
        pltpu.VMEM((_CPT, _CHUNK), jnp.int32),
        pltpu.VMEM((_CHUNK, _H), jnp.float32),
        pltpu.VMEM_SHARED((_ACC_ROWS, _H), jnp.float32),
    ],
    compiler_params=pltpu.CompilerParams(use_tc_tiling_on_sc=False),
)
def _sc_agg(y_hbm, src_hbm, dst_hbm, zero_hbm, out_hbm, src_v, dst_v, rows_v, acc):
    c = lax.axis_index("c")
    s = lax.axis_index("s")
    g = c * _NSUB + s
    pltpu.sync_copy(zero_hbm.at[pl.ds(s * _ZROWS, _ZROWS)],
                    acc.at[pl.ds(s * _ZROWS, _ZROWS)])
    pltpu.sync_copy(src_hbm.at[g], src_v)
    pltpu.sync_copy(dst_hbm.at[g], dst_v)
    plsc.subcore_barrier()

    def body(j, carry):
        pltpu.sync_copy(y_hbm.at[src_v.at[j]], rows_v)
        pltpu.sync_copy(rows_v, acc.at[dst_v.at[j]], add=True)
        return carry

    lax.fori_loop(0, _CPT, body, 0)
    plsc.subcore_barrier()
    pltpu.sync_copy(acc.at[pl.ds(s * _ZROWS, _ZROWS)],
                    out_hbm.at[c, pl.ds(s * _ZROWS, _ZROWS)])


_BN = 1000


def _mm_body(x_ref, w_ref, o_ref):
    o_ref[...] = jnp.dot(x_ref[...], w_ref[...],
                         preferred_element_type=jnp.float32)


def _mm(x, w):
    n, d = x.shape
    h = w.shape[1]
    return pl.pallas_call(
        _mm_body,
        grid=(n // _BN,),
        in_specs=[
            pl.BlockSpec((_BN, d), lambda i: (i, 0)),
            pl.BlockSpec((d, h), lambda i: (0, 0)),
        ],
        out_specs=pl.BlockSpec((_BN, h), lambda i: (i, 0)),
        out_shape=jax.ShapeDtypeStruct((n, h), jnp.float32),
    )(x, w)


def _stage_b_body(y_ref, p0_ref, p1_ref, b0_ref, w1_ref, b1_ref, w2_ref, o_ref):
    h = jnp.maximum(y_ref[...] + p0_ref[...] + p1_ref[...] + b0_ref[...], 0.0)
    t = jnp.dot(h, w1_ref[...], preferred_element_type=jnp.float32) + b1_ref[...]
    t = jnp.maximum(t, 0.0)
    o_ref[...] = jnp.dot(t, w2_ref[...], preferred_element_type=jnp.float32)


def _stage_b(y1, p0, p1, b1_0, w1_1, b1_1, w2_0):
    row = lambda i: (i, 0)
    fixed = lambda i: (0, 0)
    return pl.pallas_call(
        _stage_b_body,
        grid=(_N // _BN,),
        in_specs=[
            pl.BlockSpec((_BN, _H), row),
            pl.BlockSpec((_BN, _H), row),
            pl.BlockSpec((_BN, _H), row),
            pl.BlockSpec((1, _H), fixed),
            pl.BlockSpec((_H, _H), fixed),
            pl.BlockSpec((1, _H), fixed),
            pl.BlockSpec((_H, _H), fixed),
        ],
        out_specs=pl.BlockSpec((_BN, _H), row),
        out_shape=jax.ShapeDtypeStruct((_N, _H), jnp.float32),
    )(y1, p0, p1, b1_0.reshape(1, _H), w1_1, b1_1.reshape(1, _H), w2_0)


def _stage_c_body(y_ref, p0_ref, p1_ref, b0_ref, w1_ref, b1_ref, wf_ref,
                  bf_ref, o_ref):
    h = jnp.maximum(y_ref[...] + p0_ref[...] + p1_ref[...] + b0_ref[...], 0.0)
    t = jnp.dot(h, w1_ref[...], preferred_element_type=jnp.float32) + b1_ref[...]
    logits = jnp.dot(t, wf_ref[...], preferred_element_type=jnp.float32) + bf_ref[...]
    m = jnp.max(logits, axis=1, keepdims=True)
    lse = jnp.log(jnp.sum(jnp.exp(logits - m), axis=1, keepdims=True)) + m
    o_ref[...] = logits - lse


def _stage_c(y2, p0, p1, b2_0, w2_1, b2_1, wfc, bfc):
    row = lambda i: (i, 0)
    fixed = lambda i: (0, 0)
    return pl.pallas_call(
        _stage_c_body,
        grid=(_N // _BN,),
        in_specs=[
            pl.BlockSpec((_BN, _H), row),
            pl.BlockSpec((_BN, _H), row),
            pl.BlockSpec((_BN, _H), row),
            pl.BlockSpec((1, _H), fixed),
            pl.BlockSpec((_H, _H), fixed),
            pl.BlockSpec((1, _H), fixed),
            pl.BlockSpec((_H, _C), fixed),
            pl.BlockSpec((1, _C), fixed),
        ],
        out_specs=pl.BlockSpec((_BN, _C), row),
        out_shape=jax.ShapeDtypeStruct((_N, _C), jnp.float32),
    )(y2, p0, p1, b2_0.reshape(1, _H), w2_1, b2_1.reshape(1, _H), wfc,
      bfc.reshape(1, _C))


def kernel(x, edge_index, w1_0, b1_0, w1_1, b1_1, w2_0, b2_0, w2_1, b2_1,
           wfc, bfc):
    src = edge_index[0]
    dst = edge_index[1]
    pad = _EPAD - _E
    src_p = jnp.concatenate(
        [src, jnp.zeros((pad,), jnp.int32)]).reshape(_NTILES, _CPT, _CHUNK)
    dst_p = jnp.concatenate(
        [dst, jnp.full((pad,), _N, jnp.int32)]).reshape(_NTILES, _CPT, _CHUNK)
    zeros = jnp.zeros((_ACC_ROWS, _H), jnp.float32)

    y1 = _mm(x, w1_0)
    p = _sc_agg(y1, src_p, dst_p, zeros)
    y2 = _stage_b(y1, p[0, :_N], p[1, :_N], b1_0, w1_1, b1_1, w2_0)
    q = _sc_agg(y2, src_p, dst_p, zeros)
    return _stage_c(y2, q[0, :_N], q[1, :_N], b2_0, w2_1, b2_1, wfc, bfc)
